# baseline (speedup 1.0000x reference)
import math

import jax
import jax.numpy as jnp
from jax import lax
from jax.experimental import pallas as pl
from jax.experimental.pallas import tpu as pltpu
from jax.experimental.pallas import tpu_sc as plsc

D_MODEL = 64
SCALE = math.sqrt(D_MODEL)
LANES = 16
B, S = 4096, 200
NBJ = B // 128
NSI = S // 8
CS = 2
CHUNK = CS * 128
PITCH = 129

VOCAB = 1000000
NVT = (VOCAB + 127) // 128
VPAIR = VOCAB // 2

_info = plsc.get_sparse_core_info()
NC, NS = _info.num_cores, _info.num_subcores
NW = NC * NS


def _relay_body(lutT_hbm, out_hbm, s0, s1, d0, d1, e0, e1,
                rsem0, rsem1, wsem0, wsem1):
    wid = lax.axis_index("s") * NC + lax.axis_index("c")
    sbuf, dbuf, ebuf = (s0, s1), (d0, d1), (e0, e1)
    rsem, wsem = (rsem0, rsem1), (wsem0, wsem1)
    base = NVT // NW
    nv = jnp.where(wid < NVT - NW * base, base + 1, base)

    lane = lax.iota(jnp.int32, LANES)
    psel = [k * 8 + (lane >> 1) for k in range(8)]
    cbase = (lane & 1) * D_MODEL
    crot = lane

    def start_read(vt, b):
        pltpu.async_copy(lutT_hbm.at[:, pl.ds(vt * 128, 128)],
                         sbuf[b], rsem[b])

    def wait_read(b):
        pltpu.make_async_copy(lutT_hbm.at[:, pl.ds(0, 128)],
                              sbuf[b], rsem[b]).wait()

    def start_write(vt, b):
        @pl.when(vt < NVT - 1)
        def _():
            pltpu.async_copy(ebuf[b], out_hbm.at[pl.ds(vt * 64, 64)], wsem[b])

        @pl.when(vt == NVT - 1)
        def _():
            pltpu.async_copy(ebuf[b].at[pl.ds(0, 32)],
                             out_hbm.at[pl.ds(vt * 64, 32)], wsem[b])

    def wait_write(b):
        pltpu.make_async_copy(ebuf[b], out_hbm.at[pl.ds(0, 64)],
                              wsem[b]).wait()

    def wait_write_tail(b):
        pltpu.make_async_copy(ebuf[b].at[pl.ds(0, 32)],
                              out_hbm.at[pl.ds(0, 32)], wsem[b]).wait()

    start_read(wid, 0)

    @pl.when(nv > 1)
    def _():
        start_read(wid + NW, 1)

    def do_pair(step, carry):
        for b in (0, 1):
            i = step * 2 + b

            @pl.when(i < nv)
            def _():
                vt = wid + i * NW
                wait_read(b)

                @pl.when(i >= 2)
                def _():
                    wait_write(b)

                def trans_row(dd, c):
                    ddv = jnp.full((LANES,), dd, jnp.int32)
                    col = cbase + ((ddv + crot) & 63)
                    for k in range(8):
                        vals = sbuf[b][dd, pl.ds(k * LANES, LANES)]
                        plsc.store_scatter(dbuf[b], [psel[k], col], vals)
                    return c

                lax.fori_loop(0, D_MODEL, trans_row, 0, unroll=4)

                def fixup_row(p, c):
                    pv = jnp.full((LANES,), p, jnp.int32)
                    base = (2 * p) & 15
                    for h in (0, 1):
                        rot = (base + h) & 15
                        for m in range(4):
                            src = dbuf[b][p, pl.ds(h * 64 + m * 16, LANES)]
                            cols = h * 64 + ((lane + (m * 16 - rot)) & 63)
                            plsc.store_scatter(ebuf[b], [pv, cols], src)
                    return c

                lax.fori_loop(0, 64, fixup_row, 0, unroll=2)
                start_write(vt, b)

                @pl.when(i + 2 < nv)
                def _():
                    start_read(vt + 2 * NW, b)
        return carry

    lax.fori_loop(0, (base + 3) // 2, do_pair, 0)

    wait_write(1)
    tail_wid = (NVT - 1) % NW

    @pl.when(wid == tail_wid)
    def _():
        wait_write_tail(0)

    @pl.when(wid != tail_wid)
    def _():
        wait_write(0)


def _emb_body(table_hbm, x4_hbm, out_hbm,
              stage, g0, g1, d0, d1,
              gsem0, gsem1, wsem0, wsem1):
    bj = lax.axis_index("s") * NC + lax.axis_index("c")
    gbuf, dbuf = (g0, g1), (d0, d1)
    gsem, wsem = (gsem0, gsem1), (wsem0, wsem1)

    pltpu.sync_copy(x4_hbm.at[:, bj], stage)

    lane = lax.iota(jnp.int32, LANES)
    gsel = [(k * LANES + lane) >> 3 for k in range(D_MODEL // LANES)]
    rsel = [(k * LANES + lane) & 7 for k in range(D_MODEL // LANES)]

    def idx_slice(s):
        return stage.at[s >> 3, pl.ds((s & 7) * 128, CHUNK)]

    def start_gather(s, b):
        pltpu.async_copy(table_hbm.at[idx_slice(s)], gbuf[b], gsem[b])

    def wait_gather(b):
        pltpu.make_async_copy(
            table_hbm.at[idx_slice(0)], gbuf[b], gsem[b]).wait()

    def start_write(s, b):
        pltpu.async_copy(dbuf[b].at[:, :, :, pl.ds(0, 128)],
                         out_hbm.at[pl.ds(s, CS), :, bj], wsem[b])

    def wait_write(b):
        pltpu.make_async_copy(dbuf[b].at[:, :, :, pl.ds(0, 128)],
                              out_hbm.at[pl.ds(0, CS), :, bj], wsem[b]).wait()

    start_gather(0, 0)
    start_gather(CS, 1)

    def do_pair(step, carry):
        for b in (0, 1):
            s = (step * 2 + b) * CS
            wait_gather(b)

            @pl.when(s >= 2 * CS)
            def _():
                wait_write(b)

            for sl in range(CS):
                slv = jnp.full((LANES,), sl, jnp.int32)

                def trans_row(bp, c):
                    cv = jnp.full((LANES,), bp, jnp.int32)
                    for k in range(D_MODEL // LANES):
                        vals = gbuf[b][sl * 128 + bp, pl.ds(k * LANES, LANES)]
                        plsc.store_scatter(
                            dbuf[b], [slv, gsel[k], rsel[k], cv],
                            vals * SCALE)
                    return c

                lax.fori_loop(0, 128, trans_row, 0, unroll=4)

            start_write(s, b)

            @pl.when(s + 2 * CS < S)
            def _():
                start_gather(s + 2 * CS, b)
        return carry

    lax.fori_loop(0, S // (2 * CS), do_pair, 0)
    wait_write(0)
    wait_write(1)


def kernel(x, lut):
    x4 = (x.astype(jnp.int32).reshape(NBJ, 128, NSI, 8)
          .transpose(2, 0, 3, 1).reshape(NSI, NBJ, 1024))

    dense = pl.kernel(
        _relay_body,
        out_type=jax.ShapeDtypeStruct((VPAIR, 128), jnp.float32),
        mesh=plsc.VectorSubcoreMesh(core_axis_name="c", subcore_axis_name="s"),
        compiler_params=pltpu.CompilerParams(needs_layout_passes=False),
        scratch_types=[
            pltpu.VMEM((D_MODEL, 128), jnp.float32),
            pltpu.VMEM((D_MODEL, 128), jnp.float32),
            pltpu.VMEM((D_MODEL, 128), jnp.float32),
            pltpu.VMEM((D_MODEL, 128), jnp.float32),
            pltpu.VMEM((D_MODEL, 128), jnp.float32),
            pltpu.VMEM((D_MODEL, 128), jnp.float32),
            pltpu.SemaphoreType.DMA,
            pltpu.SemaphoreType.DMA,
            pltpu.SemaphoreType.DMA,
            pltpu.SemaphoreType.DMA,
        ],
    )(lut.T)

    table = dense.reshape(VOCAB, D_MODEL)

    out5 = pl.kernel(
        _emb_body,
        out_type=jax.ShapeDtypeStruct((S, 8, NBJ, 8, 128), jnp.float32),
        mesh=plsc.VectorSubcoreMesh(core_axis_name="c", subcore_axis_name="s"),
        compiler_params=pltpu.CompilerParams(
            use_tc_tiling_on_sc=False, needs_layout_passes=False),
        scratch_types=[
            pltpu.VMEM((NSI, 1024), jnp.int32),
            pltpu.VMEM((CHUNK, D_MODEL), jnp.float32),
            pltpu.VMEM((CHUNK, D_MODEL), jnp.float32),
            pltpu.VMEM((CS, 8, 8, PITCH), jnp.float32),
            pltpu.VMEM((CS, 8, 8, PITCH), jnp.float32),
            pltpu.SemaphoreType.DMA,
            pltpu.SemaphoreType.DMA,
            pltpu.SemaphoreType.DMA,
            pltpu.SemaphoreType.DMA,
        ],
    )(table, x4)

    o = (out5.transpose(2, 4, 0, 1, 3)
         .reshape(B, S, D_MODEL))
    return o


# speedup vs baseline: 1.2408x; 1.2408x over previous
import math

import jax
import jax.numpy as jnp
from jax import lax
from jax.experimental import pallas as pl
from jax.experimental.pallas import tpu as pltpu
from jax.experimental.pallas import tpu_sc as plsc

D_MODEL = 64
SCALE = math.sqrt(D_MODEL)
LANES = 16
B, S = 4096, 200
NBJ = B // 128
NSI = S // 8
CS = 2
CHUNK = CS * 128
PITCH = 129
GDEPTH = 4

_info = plsc.get_sparse_core_info()
NC, NS = _info.num_cores, _info.num_subcores


def _emb_body(table_hbm, x4_hbm, out_hbm,
              stage, g0, g1, g2, g3, d0, d1,
              gsem0, gsem1, gsem2, gsem3, wsem0, wsem1):
    bj = lax.axis_index("s") * NC + lax.axis_index("c")
    gbuf, dbuf = (g0, g1, g2, g3), (d0, d1)
    gsem, wsem = (gsem0, gsem1, gsem2, gsem3), (wsem0, wsem1)

    pltpu.sync_copy(x4_hbm.at[:, bj], stage)

    lane = lax.iota(jnp.int32, LANES)
    gsel = [(k * LANES + lane) >> 3 for k in range(D_MODEL // LANES)]
    rsel = [(k * LANES + lane) & 7 for k in range(D_MODEL // LANES)]

    def idx_slice(s):
        return stage.at[s >> 3, pl.ds((s & 7) * 128, CHUNK)]

    def start_gather(s, b):
        pltpu.async_copy(table_hbm.at[idx_slice(s)], gbuf[b], gsem[b])

    def wait_gather(b):
        pltpu.make_async_copy(
            table_hbm.at[idx_slice(0)], gbuf[b], gsem[b]).wait()

    def start_write(s, db):
        pltpu.async_copy(dbuf[db].at[:, :, :, pl.ds(0, 128)],
                         out_hbm.at[pl.ds(s, CS), :, bj], wsem[db])

    def wait_write(db):
        pltpu.make_async_copy(dbuf[db].at[:, :, :, pl.ds(0, 128)],
                              out_hbm.at[pl.ds(0, CS), :, bj],
                              wsem[db]).wait()

    for q in range(GDEPTH):
        start_gather(q * CS, q)

    def do_quad(step, carry):
        for b in range(GDEPTH):
            s = (step * GDEPTH + b) * CS
            db = b & 1
            wait_gather(b)

            @pl.when(s >= 2 * CS)
            def _():
                wait_write(db)

            for sl in range(CS):
                slv = jnp.full((LANES,), sl, jnp.int32)

                def trans_row(bp, c):
                    cv = jnp.full((LANES,), bp, jnp.int32)
                    for k in range(D_MODEL // LANES):
                        vals = gbuf[b][sl * 128 + bp, pl.ds(k * LANES, LANES)]
                        plsc.store_scatter(
                            dbuf[db], [slv, gsel[k], rsel[k], cv],
                            vals * SCALE)
                    return c

                lax.fori_loop(0, 128, trans_row, 0, unroll=4)

            start_write(s, db)

            @pl.when(s + GDEPTH * CS < S)
            def _():
                start_gather(s + GDEPTH * CS, b)
        return carry

    lax.fori_loop(0, S // (GDEPTH * CS), do_quad, 0)
    wait_write(0)
    wait_write(1)


def kernel(x, lut):
    x4 = (x.astype(jnp.int32).reshape(NBJ, 128, NSI, 8)
          .transpose(2, 0, 3, 1).reshape(NSI, NBJ, 1024))

    out5 = pl.kernel(
        _emb_body,
        out_type=jax.ShapeDtypeStruct((S, 8, NBJ, 8, 128), jnp.float32),
        mesh=plsc.VectorSubcoreMesh(core_axis_name="c", subcore_axis_name="s"),
        compiler_params=pltpu.CompilerParams(
            use_tc_tiling_on_sc=False, needs_layout_passes=False),
        scratch_types=[
            pltpu.VMEM((NSI, 1024), jnp.int32),
            pltpu.VMEM((CHUNK, D_MODEL), jnp.float32),
            pltpu.VMEM((CHUNK, D_MODEL), jnp.float32),
            pltpu.VMEM((CHUNK, D_MODEL), jnp.float32),
            pltpu.VMEM((CHUNK, D_MODEL), jnp.float32),
            pltpu.VMEM((CS, 8, 8, PITCH), jnp.float32),
            pltpu.VMEM((CS, 8, 8, PITCH), jnp.float32),
            pltpu.SemaphoreType.DMA,
            pltpu.SemaphoreType.DMA,
            pltpu.SemaphoreType.DMA,
            pltpu.SemaphoreType.DMA,
            pltpu.SemaphoreType.DMA,
            pltpu.SemaphoreType.DMA,
        ],
    )(lut, x4)

    o = (out5.transpose(2, 4, 0, 1, 3)
         .reshape(B, S, D_MODEL))
    return o
